# baseline (speedup 1.0000x reference)
import functools

import jax
import jax.numpy as jnp
from jax import lax
from jax.experimental import pallas as pl
from jax.experimental.pallas import tpu as pltpu
from jax.experimental.pallas import tpu_sc as plsc

B, C, N, M, K = 4, 256, 4096, 4096, 16
CG = 16
NUM_TASKS = B * (C // CG)
M_CHUNK = 1024
NUM_CHUNKS = M // M_CHUNK
GROUPS = M_CHUNK // 16


def _sc_kernel(x_hbm, idx_hbm, out_hbm, x_v, idx_v, out_v):
    nc = 2
    wid = lax.axis_index("s") * nc + lax.axis_index("c")

    iota16 = lax.iota(jnp.int32, 16)
    k_splats = [jnp.full((16,), k, jnp.int32) for k in range(K)]
    c_splats = [jnp.full((16,), c, jnp.int32) for c in range(CG)]

    def task_body(t, _):
        task = wid + 32 * t
        b = task // (C // CG)
        cg = task % (C // CG)
        c0 = cg * CG

        pltpu.sync_copy(x_hbm.at[b, pl.ds(c0, CG), :], x_v)

        def chunk_body(ch, _):
            moff = ch * M_CHUNK
            pltpu.sync_copy(idx_hbm.at[b, pl.ds(moff, M_CHUNK), :], idx_v)

            @plsc.parallel_loop(0, GROUPS, unroll=4)
            def group_body(g):
                m0 = g * 16
                rows = jnp.full((16,), m0, jnp.int32) + iota16
                gidx = [plsc.load_gather(idx_v, [rows, k_splats[k]])
                        for k in range(K)]
                for c in range(CG):
                    vals = [plsc.load_gather(x_v, [c_splats[c], gidx[k]])
                            for k in range(K)]
                    while len(vals) > 1:
                        vals = [jnp.maximum(vals[2 * i], vals[2 * i + 1])
                                for i in range(len(vals) // 2)]
                    out_v[c, pl.ds(m0, 16)] = vals[0]

            pltpu.sync_copy(
                out_v, out_hbm.at[b, pl.ds(c0, CG), pl.ds(moff, M_CHUNK)])
            return _

        lax.fori_loop(0, NUM_CHUNKS, chunk_body, None)
        return _

    lax.fori_loop(0, NUM_TASKS // 32, task_body, None)


def kernel(x, pos, support_points, indices):
    del pos, support_points
    idx = indices.astype(jnp.int32)

    mesh = plsc.VectorSubcoreMesh(core_axis_name="c", subcore_axis_name="s")
    run = functools.partial(
        pl.kernel,
        mesh=mesh,
        compiler_params=pltpu.CompilerParams(
            needs_layout_passes=False, use_tc_tiling_on_sc=False),
        out_type=jax.ShapeDtypeStruct((B, C, M), jnp.float32),
        scratch_types=[
            pltpu.VMEM((CG, N), jnp.float32),
            pltpu.VMEM((M_CHUNK, K), jnp.int32),
            pltpu.VMEM((CG, M_CHUNK), jnp.float32),
        ],
    )(_sc_kernel)
    return run(x, idx)


# speedup vs baseline: 1.4055x; 1.4055x over previous
import functools

import jax
import jax.numpy as jnp
from jax import lax
from jax.experimental import pallas as pl
from jax.experimental.pallas import tpu as pltpu
from jax.experimental.pallas import tpu_sc as plsc

B, C, N, M, K = 4, 256, 4096, 4096, 16
CG = 16
NUM_TASKS = B * (C // CG)
M_CHUNK = 1024
NUM_CHUNKS = M // M_CHUNK
GROUPS = M_CHUNK // 16
IDX_WORDS = M_CHUNK * K


def _sc_kernel(x_hbm, idx_hbm, out_hbm, x_v, idx_v, out_v,
               sem_x, sem_idx, sem_out):
    nc = 2
    wid = lax.axis_index("s") * nc + lax.axis_index("c")

    def fire_idx(b, ch, parity):
        moff = ch * M_CHUNK
        for k in range(K):
            pltpu.async_copy(
                idx_hbm.at[b, k, pl.ds(moff, M_CHUNK)],
                idx_v.at[pl.ds(parity * IDX_WORDS + k * M_CHUNK, M_CHUNK)],
                sem_idx)

    def drain_idx(b, ch, parity):
        moff = ch * M_CHUNK
        for k in range(K):
            pltpu.make_async_copy(
                idx_hbm.at[b, k, pl.ds(moff, M_CHUNK)],
                idx_v.at[pl.ds(parity * IDX_WORDS + k * M_CHUNK, M_CHUNK)],
                sem_idx).wait()

    def fire_out(b, c0, ch):
        moff = ch * M_CHUNK
        for c in range(CG):
            pltpu.async_copy(
                out_v.at[pl.ds(c * M_CHUNK, M_CHUNK)],
                out_hbm.at[b, c0 + c, pl.ds(moff, M_CHUNK)], sem_out)

    def drain_out(b, c0, ch):
        moff = ch * M_CHUNK
        for c in range(CG):
            pltpu.make_async_copy(
                out_v.at[pl.ds(c * M_CHUNK, M_CHUNK)],
                out_hbm.at[b, c0 + c, pl.ds(moff, M_CHUNK)], sem_out).wait()

    def task_body(t, _):
        task = wid + 32 * t
        b = task // (C // CG)
        cg = task % (C // CG)
        c0 = cg * CG

        for c in range(CG):
            pltpu.async_copy(x_hbm.at[b, c0 + c, :],
                             x_v.at[pl.ds(c * N, N)], sem_x)
        fire_idx(b, 0, 0)
        for c in range(CG):
            pltpu.make_async_copy(x_hbm.at[b, c0 + c, :],
                                  x_v.at[pl.ds(c * N, N)], sem_x).wait()

        def chunk_body(ch, _):
            parity = lax.rem(ch, 2)
            drain_idx(b, ch, parity)

            @pl.when(ch < NUM_CHUNKS - 1)
            def _prefetch():
                fire_idx(b, ch + 1, 1 - parity)

            @pl.when(ch > 0)
            def _drain_prev_out():
                drain_out(b, c0, ch - 1)

            pbase = parity * IDX_WORDS

            @plsc.parallel_loop(0, GROUPS, unroll=2)
            def group_body(g):
                m0 = g * 16
                gidx = [idx_v[pl.ds(pbase + k * M_CHUNK + m0, 16)]
                        for k in range(K)]
                for c in range(CG):
                    xrow = x_v.at[pl.ds(c * N, N)]
                    vals = [plsc.load_gather(xrow, [gidx[k]])
                            for k in range(K)]
                    while len(vals) > 1:
                        vals = [jnp.maximum(vals[2 * i], vals[2 * i + 1])
                                for i in range(len(vals) // 2)]
                    out_v[pl.ds(c * M_CHUNK + m0, 16)] = vals[0]

            fire_out(b, c0, ch)
            return _

        lax.fori_loop(0, NUM_CHUNKS, chunk_body, None)
        drain_out(b, c0, NUM_CHUNKS - 1)
        return _

    lax.fori_loop(0, NUM_TASKS // 32, task_body, None)


def kernel(x, pos, support_points, indices):
    del pos, support_points
    idx_t = indices.astype(jnp.int32).transpose(0, 2, 1)

    mesh = plsc.VectorSubcoreMesh(core_axis_name="c", subcore_axis_name="s")
    run = functools.partial(
        pl.kernel,
        mesh=mesh,
        compiler_params=pltpu.CompilerParams(needs_layout_passes=False),
        out_type=jax.ShapeDtypeStruct((B, C, M), jnp.float32),
        scratch_types=[
            pltpu.VMEM((CG * N,), jnp.float32),
            pltpu.VMEM((2 * IDX_WORDS,), jnp.int32),
            pltpu.VMEM((CG * M_CHUNK,), jnp.float32),
            pltpu.SemaphoreType.DMA,
            pltpu.SemaphoreType.DMA,
            pltpu.SemaphoreType.DMA,
        ],
    )(_sc_kernel)
    return run(x, idx_t)
